# baseline (speedup 1.0000x reference)
import functools

import jax
import jax.numpy as jnp
from jax import lax
from jax.experimental import pallas as pl
from jax.experimental.pallas import tpu as pltpu

EMBED = 1024
HQ = 16
HKV = 4
DH = 64
CS = 64
EPS = 1e-6
G = HQ // HKV
RQ = G * CS
UNROLL = 8


def _qproj_kernel(x_ref, nw_ref, wq_ref, o_ref, *, bq):
    x = x_ref[0]
    var = jnp.mean(x * x, axis=-1, keepdims=True)
    hh = ((x * lax.rsqrt(var + EPS)) * nw_ref[0]).astype(jnp.bfloat16)
    q = lax.dot_general(
        hh, wq_ref[...], (((1,), (0,)), ((), ())),
        preferred_element_type=jnp.float32)
    q = (q * 0.125).astype(jnp.bfloat16)
    for h in range(HKV):
        for g in range(G):
            col = (h * G + g) * DH
            for qq in range(bq // CS):
                o_ref[0, h, qq, pl.ds(g * CS, CS), :] = (
                    q[qq * CS:(qq + 1) * CS, col:col + DH])


def _oproj_kernel(ctx_ref, res_ref, wo_ref, o_ref, t_ref, *, bq):
    for h in range(HKV):
        for g in range(G):
            col = (h * G + g) * DH
            for qq in range(bq // CS):
                t_ref[qq * CS:(qq + 1) * CS, col:col + DH] = (
                    ctx_ref[0, h, qq, pl.ds(g * CS, CS), :])
    o_ref[0] = res_ref[0] + lax.dot_general(
        t_ref[...], wo_ref[...], (((1,), (0,)), ((), ())),
        preferred_element_type=jnp.float32)


def _hsa_kernel(idx_ref, w_ref, q_ref, kT_ref, v_ref, o_ref,
                ksT_ref, vs_ref, *, nqc, ksel):
    n = pl.program_id(0)
    h = pl.program_id(1)
    kc = ksel * CS

    seg = (lax.broadcasted_iota(jnp.int32, (kc, ksel), 0) // CS ==
           lax.broadcasted_iota(jnp.int32, (kc, ksel), 1)).astype(jnp.bfloat16)
    segT = (lax.broadcasted_iota(jnp.int32, (ksel, kc), 1) // CS ==
            lax.broadcasted_iota(jnp.int32, (ksel, kc), 0)).astype(jnp.bfloat16)

    def one(qc, j):
        for k in range(ksel):
            c = idx_ref[n, qc, h, k]
            ksT_ref[j, :, pl.ds(k * CS, CS)] = kT_ref[0, 0, c]
            vs_ref[j, pl.ds(k * CS, CS), :] = v_ref[0, 0, c]
        s = lax.dot_general(
            q_ref[0, 0, qc], ksT_ref[j], (((1,), (0,)), ((), ())),
            preferred_element_type=jnp.float32)
        m = jnp.max(s, axis=-1, keepdims=True)
        e = jnp.exp(s - m)
        d8 = lax.dot_general(
            e.astype(jnp.bfloat16), seg, (((1,), (0,)), ((), ())),
            preferred_element_type=jnp.float32)
        w8 = jnp.concatenate(
            [w_ref[n, qc, h, k].reshape(1, 1) for k in range(ksel)], axis=1)
        rx = lax.dot_general(
            (w8 / d8).astype(jnp.bfloat16), segT, (((1,), (0,)), ((), ())),
            preferred_element_type=jnp.float32)
        p = (e * rx).astype(jnp.bfloat16)
        out = lax.dot_general(
            p, vs_ref[j], (((1,), (0,)), ((), ())),
            preferred_element_type=jnp.float32)
        o_ref[0, 0, qc] = out.astype(jnp.bfloat16)

    def body(i, carry):
        for j in range(UNROLL):
            one(UNROLL * i + j, j)
        return carry

    lax.fori_loop(0, nqc // UNROLL, body, 0)


def kernel(hidden_states, weights, mem_k, mem_v, landmarks, indices, norm_w,
           Wq, Wo):
    N, L, _ = hidden_states.shape
    KVLEN = mem_k.shape[1]
    NQC = L // CS
    K = indices.shape[-1]

    BQ = 512
    q5 = pl.pallas_call(
        functools.partial(_qproj_kernel, bq=BQ),
        grid=(N, L // BQ),
        in_specs=[
            pl.BlockSpec((1, BQ, EMBED), lambda n, i: (n, i, 0)),
            pl.BlockSpec((1, EMBED), lambda n, i: (0, 0)),
            pl.BlockSpec((EMBED, EMBED), lambda n, i: (0, 0)),
        ],
        out_specs=pl.BlockSpec((1, HKV, BQ // CS, RQ, DH),
                               lambda n, i: (n, 0, i, 0, 0)),
        out_shape=jax.ShapeDtypeStruct((N, HKV, NQC, RQ, DH), jnp.bfloat16),
        compiler_params=pltpu.CompilerParams(
            dimension_semantics=("parallel", "parallel")),
    )(hidden_states, norm_w.reshape(1, EMBED), Wq.T.astype(jnp.bfloat16))

    C = KVLEN // CS
    ktT = mem_k.astype(jnp.bfloat16).reshape(
        N, C, CS, HKV, DH).transpose(0, 3, 1, 4, 2)
    vt = mem_v.astype(jnp.bfloat16).reshape(
        N, C, CS, HKV, DH).transpose(0, 3, 1, 2, 4)

    hsa = functools.partial(_hsa_kernel, nqc=NQC, ksel=K)
    ctx = pl.pallas_call(
        hsa,
        grid_spec=pltpu.PrefetchScalarGridSpec(
            num_scalar_prefetch=2,
            grid=(N, HKV),
            in_specs=[
                pl.BlockSpec((1, 1, NQC, RQ, DH),
                             lambda n, h, idx, w: (n, h, 0, 0, 0)),
                pl.BlockSpec((1, 1, C, DH, CS),
                             lambda n, h, idx, w: (n, h, 0, 0, 0)),
                pl.BlockSpec((1, 1, C, CS, DH),
                             lambda n, h, idx, w: (n, h, 0, 0, 0)),
            ],
            out_specs=pl.BlockSpec((1, 1, NQC, RQ, DH),
                                   lambda n, h, idx, w: (n, h, 0, 0, 0)),
            scratch_shapes=[
                pltpu.VMEM((UNROLL, DH, K * CS), jnp.bfloat16),
                pltpu.VMEM((UNROLL, K * CS, DH), jnp.bfloat16),
            ],
        ),
        out_shape=jax.ShapeDtypeStruct((N, HKV, NQC, RQ, DH), jnp.bfloat16),
        compiler_params=pltpu.CompilerParams(
            dimension_semantics=("parallel", "parallel")),
    )(indices, weights, q5, ktT, vt)

    out = pl.pallas_call(
        functools.partial(_oproj_kernel, bq=BQ),
        grid=(N, L // BQ),
        in_specs=[
            pl.BlockSpec((1, HKV, BQ // CS, RQ, DH),
                         lambda n, i: (n, 0, i, 0, 0)),
            pl.BlockSpec((1, BQ, EMBED), lambda n, i: (n, i, 0)),
            pl.BlockSpec((EMBED, EMBED), lambda n, i: (0, 0)),
        ],
        out_specs=pl.BlockSpec((1, BQ, EMBED), lambda n, i: (n, i, 0)),
        out_shape=jax.ShapeDtypeStruct((N, L, EMBED), jnp.float32),
        scratch_shapes=[pltpu.VMEM((BQ, EMBED), jnp.bfloat16)],
        compiler_params=pltpu.CompilerParams(
            dimension_semantics=("parallel", "parallel")),
    )(ctx, hidden_states, Wo.T.astype(jnp.bfloat16))

    return (out, weights, mem_k, mem_v, landmarks, indices)


# speedup vs baseline: 1.0203x; 1.0203x over previous
import functools

import jax
import jax.numpy as jnp
from jax import lax
from jax.experimental import pallas as pl
from jax.experimental.pallas import tpu as pltpu

EMBED = 1024
HQ = 16
HKV = 4
DH = 64
CS = 64
EPS = 1e-6
G = HQ // HKV
RQ = G * CS
UNROLL = 16


def _qproj_kernel(x_ref, nw_ref, wq_ref, o_ref, *, bq):
    x = x_ref[0]
    var = jnp.mean(x * x, axis=-1, keepdims=True)
    hh = ((x * lax.rsqrt(var + EPS)) * nw_ref[0]).astype(jnp.bfloat16)
    q = lax.dot_general(
        hh, wq_ref[...], (((1,), (0,)), ((), ())),
        preferred_element_type=jnp.float32)
    q = (q * 0.125).astype(jnp.bfloat16)
    for h in range(HKV):
        for g in range(G):
            col = (h * G + g) * DH
            for qq in range(bq // CS):
                o_ref[0, h, qq, pl.ds(g * CS, CS), :] = (
                    q[qq * CS:(qq + 1) * CS, col:col + DH])


def _oproj_kernel(ctx_ref, res_ref, wo_ref, o_ref, t_ref, *, bq):
    for h in range(HKV):
        for g in range(G):
            col = (h * G + g) * DH
            for qq in range(bq // CS):
                t_ref[qq * CS:(qq + 1) * CS, col:col + DH] = (
                    ctx_ref[0, h, qq, pl.ds(g * CS, CS), :])
    o_ref[0] = res_ref[0] + lax.dot_general(
        t_ref[...], wo_ref[...], (((1,), (0,)), ((), ())),
        preferred_element_type=jnp.float32)


def _hsa_kernel(idx_ref, w_ref, q_ref, kT_ref, v_ref, o_ref,
                ksT_ref, vs_ref, *, nqc, ksel):
    n = pl.program_id(0)
    h = pl.program_id(1)
    kc = ksel * CS

    seg = (lax.broadcasted_iota(jnp.int32, (kc, ksel), 0) // CS ==
           lax.broadcasted_iota(jnp.int32, (kc, ksel), 1)).astype(jnp.bfloat16)
    segT = (lax.broadcasted_iota(jnp.int32, (ksel, kc), 1) // CS ==
            lax.broadcasted_iota(jnp.int32, (ksel, kc), 0)).astype(jnp.bfloat16)

    def one(qc, j):
        for k in range(ksel):
            c = idx_ref[n, qc, h, k]
            ksT_ref[j, :, pl.ds(k * CS, CS)] = kT_ref[0, 0, c]
            vs_ref[j, pl.ds(k * CS, CS), :] = v_ref[0, 0, c]
        s = lax.dot_general(
            q_ref[0, 0, qc], ksT_ref[j], (((1,), (0,)), ((), ())),
            preferred_element_type=jnp.float32)
        m = jnp.max(s, axis=-1, keepdims=True)
        e = jnp.exp(s - m)
        d8 = lax.dot_general(
            e.astype(jnp.bfloat16), seg, (((1,), (0,)), ((), ())),
            preferred_element_type=jnp.float32)
        w8 = jnp.concatenate(
            [w_ref[n, qc, h, k].reshape(1, 1) for k in range(ksel)], axis=1)
        rx = lax.dot_general(
            (w8 / d8).astype(jnp.bfloat16), segT, (((1,), (0,)), ((), ())),
            preferred_element_type=jnp.float32)
        p = (e * rx).astype(jnp.bfloat16)
        out = lax.dot_general(
            p, vs_ref[j], (((1,), (0,)), ((), ())),
            preferred_element_type=jnp.float32)
        o_ref[0, 0, qc] = out.astype(jnp.bfloat16)

    def body(i, carry):
        for j in range(UNROLL):
            one(UNROLL * i + j, j)
        return carry

    lax.fori_loop(0, nqc // UNROLL, body, 0)


def kernel(hidden_states, weights, mem_k, mem_v, landmarks, indices, norm_w,
           Wq, Wo):
    N, L, _ = hidden_states.shape
    KVLEN = mem_k.shape[1]
    NQC = L // CS
    K = indices.shape[-1]

    BQ = 512
    q5 = pl.pallas_call(
        functools.partial(_qproj_kernel, bq=BQ),
        grid=(N, L // BQ),
        in_specs=[
            pl.BlockSpec((1, BQ, EMBED), lambda n, i: (n, i, 0)),
            pl.BlockSpec((1, EMBED), lambda n, i: (0, 0)),
            pl.BlockSpec((EMBED, EMBED), lambda n, i: (0, 0)),
        ],
        out_specs=pl.BlockSpec((1, HKV, BQ // CS, RQ, DH),
                               lambda n, i: (n, 0, i, 0, 0)),
        out_shape=jax.ShapeDtypeStruct((N, HKV, NQC, RQ, DH), jnp.bfloat16),
        compiler_params=pltpu.CompilerParams(
            dimension_semantics=("parallel", "parallel")),
    )(hidden_states, norm_w.reshape(1, EMBED), Wq.T.astype(jnp.bfloat16))

    C = KVLEN // CS
    ktT = mem_k.astype(jnp.bfloat16).reshape(
        N, C, CS, HKV, DH).transpose(0, 3, 1, 4, 2)
    vt = mem_v.astype(jnp.bfloat16).reshape(
        N, C, CS, HKV, DH).transpose(0, 3, 1, 2, 4)

    hsa = functools.partial(_hsa_kernel, nqc=NQC, ksel=K)
    ctx = pl.pallas_call(
        hsa,
        grid_spec=pltpu.PrefetchScalarGridSpec(
            num_scalar_prefetch=2,
            grid=(N, HKV),
            in_specs=[
                pl.BlockSpec((1, 1, NQC, RQ, DH),
                             lambda n, h, idx, w: (n, h, 0, 0, 0)),
                pl.BlockSpec((1, 1, C, DH, CS),
                             lambda n, h, idx, w: (n, h, 0, 0, 0)),
                pl.BlockSpec((1, 1, C, CS, DH),
                             lambda n, h, idx, w: (n, h, 0, 0, 0)),
            ],
            out_specs=pl.BlockSpec((1, 1, NQC, RQ, DH),
                                   lambda n, h, idx, w: (n, h, 0, 0, 0)),
            scratch_shapes=[
                pltpu.VMEM((UNROLL, DH, K * CS), jnp.bfloat16),
                pltpu.VMEM((UNROLL, K * CS, DH), jnp.bfloat16),
            ],
        ),
        out_shape=jax.ShapeDtypeStruct((N, HKV, NQC, RQ, DH), jnp.bfloat16),
        compiler_params=pltpu.CompilerParams(
            dimension_semantics=("parallel", "parallel")),
    )(indices, weights, q5, ktT, vt)

    out = pl.pallas_call(
        functools.partial(_oproj_kernel, bq=BQ),
        grid=(N, L // BQ),
        in_specs=[
            pl.BlockSpec((1, HKV, BQ // CS, RQ, DH),
                         lambda n, i: (n, 0, i, 0, 0)),
            pl.BlockSpec((1, BQ, EMBED), lambda n, i: (n, i, 0)),
            pl.BlockSpec((EMBED, EMBED), lambda n, i: (0, 0)),
        ],
        out_specs=pl.BlockSpec((1, BQ, EMBED), lambda n, i: (n, i, 0)),
        out_shape=jax.ShapeDtypeStruct((N, L, EMBED), jnp.float32),
        scratch_shapes=[pltpu.VMEM((BQ, EMBED), jnp.bfloat16)],
        compiler_params=pltpu.CompilerParams(
            dimension_semantics=("parallel", "parallel")),
    )(ctx, hidden_states, Wo.T.astype(jnp.bfloat16))

    return (out, weights, mem_k, mem_v, landmarks, indices)


# speedup vs baseline: 1.0688x; 1.0475x over previous
import functools

import jax
import jax.numpy as jnp
from jax import lax
from jax.experimental import pallas as pl
from jax.experimental.pallas import tpu as pltpu

EMBED = 1024
HQ = 16
HKV = 4
DH = 64
CS = 64
EPS = 1e-6
G = HQ // HKV
RQ = G * CS
UNROLL = 16


def _qproj_kernel(x_ref, nw_ref, wq_ref, o_ref, *, bq):
    x = x_ref[0]
    var = jnp.mean(x * x, axis=-1, keepdims=True)
    hh = ((x * lax.rsqrt(var + EPS)) * nw_ref[0]).astype(jnp.bfloat16)
    q = lax.dot_general(
        hh, wq_ref[...], (((1,), (0,)), ((), ())),
        preferred_element_type=jnp.float32)
    q = (q * 0.125).astype(jnp.bfloat16)
    for h in range(HKV):
        for g in range(G):
            col = (h * G + g) * DH
            for qq in range(bq // CS):
                o_ref[0, h, qq, pl.ds(g * CS, CS), :] = (
                    q[qq * CS:(qq + 1) * CS, col:col + DH])


def _oproj_kernel(ctx_ref, res_ref, wo_ref, o_ref, t_ref, *, bq):
    for h in range(HKV):
        for g in range(G):
            col = (h * G + g) * DH
            for qq in range(bq // CS):
                t_ref[qq * CS:(qq + 1) * CS, col:col + DH] = (
                    ctx_ref[0, h, qq, pl.ds(g * CS, CS), :])
    o_ref[0] = res_ref[0] + lax.dot_general(
        t_ref[...], wo_ref[...], (((1,), (0,)), ((), ())),
        preferred_element_type=jnp.float32)


def _hsa_kernel(idx_ref, w_ref, q_ref, k_ref, v_ref, o_ref,
                ksT_ref, vs_ref, kT_ref, *, nqc, ksel, nc):
    n = pl.program_id(0)
    h = pl.program_id(1)
    kc = ksel * CS

    kT_ref[...] = jnp.swapaxes(k_ref[0, 0], 1, 2)

    seg = (lax.broadcasted_iota(jnp.int32, (kc, ksel), 0) // CS ==
           lax.broadcasted_iota(jnp.int32, (kc, ksel), 1)).astype(jnp.bfloat16)
    segT = (lax.broadcasted_iota(jnp.int32, (ksel, kc), 1) // CS ==
            lax.broadcasted_iota(jnp.int32, (ksel, kc), 0)).astype(jnp.bfloat16)

    def one(qc, j):
        for k in range(ksel):
            c = idx_ref[n, qc, h, k]
            ksT_ref[j, :, pl.ds(k * CS, CS)] = kT_ref[c]
            vs_ref[j, pl.ds(k * CS, CS), :] = v_ref[0, 0, c]
        s = lax.dot_general(
            q_ref[0, 0, qc], ksT_ref[j], (((1,), (0,)), ((), ())),
            preferred_element_type=jnp.float32)
        m = jnp.max(s, axis=-1, keepdims=True)
        e = jnp.exp(s - m)
        d8 = lax.dot_general(
            e.astype(jnp.bfloat16), seg, (((1,), (0,)), ((), ())),
            preferred_element_type=jnp.float32)
        w8 = jnp.concatenate(
            [w_ref[n, qc, h, k].reshape(1, 1) for k in range(ksel)], axis=1)
        rx = lax.dot_general(
            (w8 / d8).astype(jnp.bfloat16), segT, (((1,), (0,)), ((), ())),
            preferred_element_type=jnp.float32)
        p = (e * rx).astype(jnp.bfloat16)
        out = lax.dot_general(
            p, vs_ref[j], (((1,), (0,)), ((), ())),
            preferred_element_type=jnp.float32)
        o_ref[0, 0, qc] = out.astype(jnp.bfloat16)

    def body(i, carry):
        for j in range(UNROLL):
            one(UNROLL * i + j, j)
        return carry

    lax.fori_loop(0, nqc // UNROLL, body, 0)


def kernel(hidden_states, weights, mem_k, mem_v, landmarks, indices, norm_w,
           Wq, Wo):
    N, L, _ = hidden_states.shape
    KVLEN = mem_k.shape[1]
    NQC = L // CS
    K = indices.shape[-1]

    BQ = 512
    q5 = pl.pallas_call(
        functools.partial(_qproj_kernel, bq=BQ),
        grid=(N, L // BQ),
        in_specs=[
            pl.BlockSpec((1, BQ, EMBED), lambda n, i: (n, i, 0)),
            pl.BlockSpec((1, EMBED), lambda n, i: (0, 0)),
            pl.BlockSpec((EMBED, EMBED), lambda n, i: (0, 0)),
        ],
        out_specs=pl.BlockSpec((1, HKV, BQ // CS, RQ, DH),
                               lambda n, i: (n, 0, i, 0, 0)),
        out_shape=jax.ShapeDtypeStruct((N, HKV, NQC, RQ, DH), jnp.bfloat16),
        compiler_params=pltpu.CompilerParams(
            dimension_semantics=("parallel", "parallel")),
    )(hidden_states, norm_w.reshape(1, EMBED), Wq.T.astype(jnp.bfloat16))

    C = KVLEN // CS
    kt = mem_k.astype(jnp.bfloat16).reshape(
        N, C, CS, HKV, DH).transpose(0, 3, 1, 2, 4)
    vt = mem_v.astype(jnp.bfloat16).reshape(
        N, C, CS, HKV, DH).transpose(0, 3, 1, 2, 4)

    hsa = functools.partial(_hsa_kernel, nqc=NQC, ksel=K, nc=C)
    ctx = pl.pallas_call(
        hsa,
        grid_spec=pltpu.PrefetchScalarGridSpec(
            num_scalar_prefetch=2,
            grid=(N, HKV),
            in_specs=[
                pl.BlockSpec((1, 1, NQC, RQ, DH),
                             lambda n, h, idx, w: (n, h, 0, 0, 0)),
                pl.BlockSpec((1, 1, C, CS, DH),
                             lambda n, h, idx, w: (n, h, 0, 0, 0)),
                pl.BlockSpec((1, 1, C, CS, DH),
                             lambda n, h, idx, w: (n, h, 0, 0, 0)),
            ],
            out_specs=pl.BlockSpec((1, 1, NQC, RQ, DH),
                                   lambda n, h, idx, w: (n, h, 0, 0, 0)),
            scratch_shapes=[
                pltpu.VMEM((UNROLL, DH, K * CS), jnp.bfloat16),
                pltpu.VMEM((UNROLL, K * CS, DH), jnp.bfloat16),
                pltpu.VMEM((C, DH, CS), jnp.bfloat16),
            ],
        ),
        out_shape=jax.ShapeDtypeStruct((N, HKV, NQC, RQ, DH), jnp.bfloat16),
        compiler_params=pltpu.CompilerParams(
            dimension_semantics=("parallel", "parallel")),
    )(indices, weights, q5, kt, vt)

    out = pl.pallas_call(
        functools.partial(_oproj_kernel, bq=BQ),
        grid=(N, L // BQ),
        in_specs=[
            pl.BlockSpec((1, HKV, BQ // CS, RQ, DH),
                         lambda n, i: (n, 0, i, 0, 0)),
            pl.BlockSpec((1, BQ, EMBED), lambda n, i: (n, i, 0)),
            pl.BlockSpec((EMBED, EMBED), lambda n, i: (0, 0)),
        ],
        out_specs=pl.BlockSpec((1, BQ, EMBED), lambda n, i: (n, i, 0)),
        out_shape=jax.ShapeDtypeStruct((N, L, EMBED), jnp.float32),
        scratch_shapes=[pltpu.VMEM((BQ, EMBED), jnp.bfloat16)],
        compiler_params=pltpu.CompilerParams(
            dimension_semantics=("parallel", "parallel")),
    )(ctx, hidden_states, Wo.T.astype(jnp.bfloat16))

    return (out, weights, mem_k, mem_v, landmarks, indices)


# speedup vs baseline: 1.0839x; 1.0141x over previous
import functools

import jax
import jax.numpy as jnp
from jax import lax
from jax.experimental import pallas as pl
from jax.experimental.pallas import tpu as pltpu

EMBED = 1024
HQ = 16
HKV = 4
DH = 64
CS = 64
EPS = 1e-6
G = HQ // HKV
RQ = G * CS
UNROLL = 16


def _qproj_kernel(x_ref, nw_ref, wq_ref, o_ref, *, bq):
    x = x_ref[0]
    var = jnp.mean(x * x, axis=-1, keepdims=True)
    hh = ((x * lax.rsqrt(var + EPS)) * nw_ref[0]).astype(jnp.bfloat16)
    q = lax.dot_general(
        hh, wq_ref[...], (((1,), (0,)), ((), ())),
        preferred_element_type=jnp.float32)
    q = (q * 0.125).astype(jnp.bfloat16)
    for h in range(HKV):
        for g in range(G):
            col = (h * G + g) * DH
            for qq in range(bq // CS):
                o_ref[0, h, qq, pl.ds(g * CS, CS), :] = (
                    q[qq * CS:(qq + 1) * CS, col:col + DH])


def _oproj_kernel(ctx_ref, res_ref, wo_ref, o_ref, t_ref, *, bq):
    for h in range(HKV):
        for g in range(G):
            col = (h * G + g) * DH
            for qq in range(bq // CS):
                t_ref[qq * CS:(qq + 1) * CS, col:col + DH] = (
                    ctx_ref[0, h, qq, pl.ds(g * CS, CS), :])
    o_ref[0] = res_ref[0] + lax.dot_general(
        t_ref[...], wo_ref[...], (((1,), (0,)), ((), ())),
        preferred_element_type=jnp.float32)


def _hsa_kernel(idx_ref, w_ref, q_ref, k_ref, v_ref, o_ref,
                ksT_ref, vs_ref, kT_ref, *, nqc, ksel, nc):
    n = pl.program_id(0)
    h = pl.program_id(1)
    kc = ksel * CS

    kT_ref[...] = jnp.swapaxes(k_ref[0, 0], 1, 2)

    seg = (lax.broadcasted_iota(jnp.int32, (kc, ksel), 0) // CS ==
           lax.broadcasted_iota(jnp.int32, (kc, ksel), 1)).astype(jnp.bfloat16)
    segT = (lax.broadcasted_iota(jnp.int32, (ksel, kc), 1) // CS ==
            lax.broadcasted_iota(jnp.int32, (ksel, kc), 0)).astype(jnp.bfloat16)

    def one(qc, j):
        for k in range(ksel):
            c = idx_ref[n, qc, h, k]
            ksT_ref[j, :, pl.ds(k * CS, CS)] = kT_ref[c]
            vs_ref[j, pl.ds(k * CS, CS), :] = v_ref[0, 0, c]
        s = lax.dot_general(
            q_ref[0, 0, qc], ksT_ref[j], (((1,), (0,)), ((), ())),
            preferred_element_type=jnp.float32)
        m = jnp.max(s, axis=-1, keepdims=True)
        e = jnp.exp(s - m)
        d8 = lax.dot_general(
            e.astype(jnp.bfloat16), seg, (((1,), (0,)), ((), ())),
            preferred_element_type=jnp.float32)
        w8 = jnp.concatenate(
            [w_ref[n, qc, h, k].reshape(1, 1) for k in range(ksel)], axis=1)
        rx = lax.dot_general(
            (w8 / d8).astype(jnp.bfloat16), segT, (((1,), (0,)), ((), ())),
            preferred_element_type=jnp.float32)
        p = (e * rx).astype(jnp.bfloat16)
        out = lax.dot_general(
            p, vs_ref[j], (((1,), (0,)), ((), ())),
            preferred_element_type=jnp.float32)
        o_ref[0, 0, qc] = out.astype(jnp.bfloat16)

    def body(i, carry):
        for j in range(UNROLL):
            one(UNROLL * i + j, j)
        return carry

    lax.fori_loop(0, nqc // UNROLL, body, 0)


def kernel(hidden_states, weights, mem_k, mem_v, landmarks, indices, norm_w,
           Wq, Wo):
    N, L, _ = hidden_states.shape
    KVLEN = mem_k.shape[1]
    NQC = L // CS
    K = indices.shape[-1]

    BQ = 1024
    q5 = pl.pallas_call(
        functools.partial(_qproj_kernel, bq=BQ),
        grid=(N, L // BQ),
        in_specs=[
            pl.BlockSpec((1, BQ, EMBED), lambda n, i: (n, i, 0)),
            pl.BlockSpec((1, EMBED), lambda n, i: (0, 0)),
            pl.BlockSpec((EMBED, EMBED), lambda n, i: (0, 0)),
        ],
        out_specs=pl.BlockSpec((1, HKV, BQ // CS, RQ, DH),
                               lambda n, i: (n, 0, i, 0, 0)),
        out_shape=jax.ShapeDtypeStruct((N, HKV, NQC, RQ, DH), jnp.bfloat16),
        compiler_params=pltpu.CompilerParams(
            dimension_semantics=("parallel", "parallel")),
    )(hidden_states, norm_w.reshape(1, EMBED), Wq.T.astype(jnp.bfloat16))

    C = KVLEN // CS
    kt = mem_k.astype(jnp.bfloat16).reshape(
        N, C, CS, HKV, DH).transpose(0, 3, 1, 2, 4)
    vt = mem_v.astype(jnp.bfloat16).reshape(
        N, C, CS, HKV, DH).transpose(0, 3, 1, 2, 4)

    hsa = functools.partial(_hsa_kernel, nqc=NQC, ksel=K, nc=C)
    ctx = pl.pallas_call(
        hsa,
        grid_spec=pltpu.PrefetchScalarGridSpec(
            num_scalar_prefetch=2,
            grid=(N, HKV),
            in_specs=[
                pl.BlockSpec((1, 1, NQC, RQ, DH),
                             lambda n, h, idx, w: (n, h, 0, 0, 0)),
                pl.BlockSpec((1, 1, C, CS, DH),
                             lambda n, h, idx, w: (n, h, 0, 0, 0)),
                pl.BlockSpec((1, 1, C, CS, DH),
                             lambda n, h, idx, w: (n, h, 0, 0, 0)),
            ],
            out_specs=pl.BlockSpec((1, 1, NQC, RQ, DH),
                                   lambda n, h, idx, w: (n, h, 0, 0, 0)),
            scratch_shapes=[
                pltpu.VMEM((UNROLL, DH, K * CS), jnp.bfloat16),
                pltpu.VMEM((UNROLL, K * CS, DH), jnp.bfloat16),
                pltpu.VMEM((C, DH, CS), jnp.bfloat16),
            ],
        ),
        out_shape=jax.ShapeDtypeStruct((N, HKV, NQC, RQ, DH), jnp.bfloat16),
        compiler_params=pltpu.CompilerParams(
            dimension_semantics=("parallel", "parallel")),
    )(indices, weights, q5, kt, vt)

    out = pl.pallas_call(
        functools.partial(_oproj_kernel, bq=BQ),
        grid=(N, L // BQ),
        in_specs=[
            pl.BlockSpec((1, HKV, BQ // CS, RQ, DH),
                         lambda n, i: (n, 0, i, 0, 0)),
            pl.BlockSpec((1, BQ, EMBED), lambda n, i: (n, i, 0)),
            pl.BlockSpec((EMBED, EMBED), lambda n, i: (0, 0)),
        ],
        out_specs=pl.BlockSpec((1, BQ, EMBED), lambda n, i: (n, i, 0)),
        out_shape=jax.ShapeDtypeStruct((N, L, EMBED), jnp.float32),
        scratch_shapes=[pltpu.VMEM((BQ, EMBED), jnp.bfloat16)],
        compiler_params=pltpu.CompilerParams(
            dimension_semantics=("parallel", "parallel")),
    )(ctx, hidden_states, Wo.T.astype(jnp.bfloat16))

    return (out, weights, mem_k, mem_v, landmarks, indices)
